# baseline (speedup 1.0000x reference)
import functools

import jax
import jax.numpy as jnp
from jax import lax
from jax.experimental import pallas as pl
from jax.experimental.pallas import tpu as pltpu
from jax.experimental.pallas import tpu_sc as plsc

NC = 2
NS = 16
NW = NC * NS

CH = 128
NB = 8


PAIR_W = 16384


def _transpose_table_tc(table_t, v, d):
    w = PAIR_W
    grid = (v + 2 * w - 1) // (2 * w)
    h = grid * w

    def body(in_ref, out_ref):
        x = in_ref[...]
        out_ref[...] = jnp.concatenate([x[:, :w].T, x[:, w:].T], axis=1)

    return pl.pallas_call(
        body,
        grid=(grid,),
        in_specs=[pl.BlockSpec((d, 2 * w), lambda i: (0, i))],
        out_specs=pl.BlockSpec((w, 2 * d), lambda i: (i, 0)),
        out_shape=jax.ShapeDtypeStruct((h, 2 * d), jnp.float32),
    )(table_t), h


@functools.partial(jax.jit, static_argnums=(2, 3, 4, 5))
def _emb_lookup(xr, table, total, d, nch, half):
    mesh = plsc.VectorSubcoreMesh(core_axis_name="c", subcore_axis_name="s")
    b_per_w = nch * CH

    @functools.partial(
        pl.kernel,
        mesh=mesh,
        out_type=jax.ShapeDtypeStruct((total, d), jnp.float32),
        scratch_types=[
            pltpu.VMEM((nch, CH), jnp.int32),
            pltpu.VMEM((NB, CH, d), jnp.float32),
            pltpu.SemaphoreType.DMA,
            pltpu.SemaphoreType.DMA,
        ],
        compiler_params=pltpu.CompilerParams(use_tc_tiling_on_sc=False),
    )
    def k(x_hbm, tab_hbm, out_hbm, idx_v, rows_v, gsem, ssem):
        wid = lax.axis_index("s") * NC + lax.axis_index("c")
        base = wid * b_per_w
        pltpu.sync_copy(x_hbm.at[wid], idx_v)

        @pl.loop(0, nch)
        def _(c):
            row = idx_v.at[c]
            for g in range(CH // 16):
                vv = row[pl.ds(g * 16, 16)]
                t1 = (vv >> 15) << 15
                t2 = (vv & (PAIR_W - 1)) << 1
                t3 = (vv >> 14) & 1
                row[pl.ds(g * 16, 16)] = t1 + t2 + t3

        def gather(c, b):
            pltpu.async_copy(tab_hbm.at[idx_v.at[c]], rows_v.at[b], gsem)

        def wait_gather(b):
            pltpu.make_async_copy(
                tab_hbm.at[idx_v.at[0]], rows_v.at[b], gsem).wait()

        def wait_scatter():
            pltpu.make_async_copy(
                rows_v.at[0], out_hbm.at[pl.ds(base, CH)], ssem).wait()

        for b in range(NB):
            gather(b, b)

        @pl.loop(0, nch // NB)
        def _(p):
            for b in range(NB):
                s = p * NB + b
                wait_gather(b)
                pltpu.async_copy(
                    rows_v.at[b], out_hbm.at[pl.ds(base + s * CH, CH)], ssem)
                @pl.when(jnp.logical_and(s >= 2, s < nch - NB + 2))
                def _():
                    wait_scatter()
                    gather(s + NB - 2, (b - 2) % NB)

        for _ in range(NB):
            wait_scatter()

    return k(xr, table)


def kernel(x, embeddings):
    b, h = x.shape
    _, d = embeddings.shape
    total = b * h
    b_per_w = total // NW
    nch = b_per_w // CH
    xr = x.reshape(NW, nch, CH).astype(jnp.int32)
    v = embeddings.shape[0]
    table_lin, hh = _transpose_table_tc(embeddings.T, v, d)
    out = _emb_lookup(xr, table_lin.reshape(2 * hh, d), total, d, nch, hh)
    return out.reshape(b, h, d)


# speedup vs baseline: 1.0003x; 1.0003x over previous
import functools

import jax
import jax.numpy as jnp
from jax import lax
from jax.experimental import pallas as pl
from jax.experimental.pallas import tpu as pltpu
from jax.experimental.pallas import tpu_sc as plsc

NC = 2
NS = 16
NW = NC * NS

CH = 128
NB = 10


PAIR_W = 16384


def _transpose_table_tc(table_t, v, d):
    w = PAIR_W
    grid = (v + 2 * w - 1) // (2 * w)
    h = grid * w

    def body(in_ref, out_ref):
        x = in_ref[...]
        out_ref[...] = jnp.concatenate([x[:, :w].T, x[:, w:].T], axis=1)

    return pl.pallas_call(
        body,
        grid=(grid,),
        in_specs=[pl.BlockSpec((d, 2 * w), lambda i: (0, i))],
        out_specs=pl.BlockSpec((w, 2 * d), lambda i: (i, 0)),
        out_shape=jax.ShapeDtypeStruct((h, 2 * d), jnp.float32),
    )(table_t), h


@functools.partial(jax.jit, static_argnums=(2, 3, 4))
def _emb_lookup(xr, table, total, d, nch):
    mesh = plsc.VectorSubcoreMesh(core_axis_name="c", subcore_axis_name="s")
    b_per_w = nch * CH

    @functools.partial(
        pl.kernel,
        mesh=mesh,
        out_type=jax.ShapeDtypeStruct((total, d), jnp.float32),
        scratch_types=[
            pltpu.VMEM((nch, CH), jnp.int32),
            pltpu.VMEM((NB, CH, d), jnp.float32),
            pltpu.SemaphoreType.DMA,
            pltpu.SemaphoreType.DMA,
        ],
        compiler_params=pltpu.CompilerParams(use_tc_tiling_on_sc=False),
    )
    def k(x_hbm, tab_hbm, out_hbm, idx_v, rows_v, gsem, ssem):
        wid = lax.axis_index("s") * NC + lax.axis_index("c")
        base = wid * b_per_w
        pltpu.sync_copy(x_hbm.at[wid], idx_v)

        @pl.loop(0, nch)
        def _(c):
            row = idx_v.at[c]
            for g in range(CH // 16):
                vv = row[pl.ds(g * 16, 16)]
                t1 = (vv >> 15) << 15
                t2 = (vv & (PAIR_W - 1)) << 1
                t3 = (vv >> 14) & 1
                row[pl.ds(g * 16, 16)] = t1 + t2 + t3

        def gather(c, b):
            pltpu.async_copy(tab_hbm.at[idx_v.at[c]], rows_v.at[b], gsem)

        def wait_gather(b):
            pltpu.make_async_copy(
                tab_hbm.at[idx_v.at[0]], rows_v.at[b], gsem).wait()

        def wait_scatter():
            pltpu.make_async_copy(
                rows_v.at[0], out_hbm.at[pl.ds(base, CH)], ssem).wait()

        for b in range(NB):
            gather(b, b)

        @pl.loop(0, nch // NB)
        def _(p):
            for b in range(NB):
                s = p * NB + b
                wait_gather(b)
                pltpu.async_copy(
                    rows_v.at[b], out_hbm.at[pl.ds(base + s * CH, CH)], ssem)
                @pl.when(jnp.logical_and(s >= 2, s < nch - NB + 2))
                def _():
                    wait_scatter()
                    gather(s + NB - 2, (b - 2) % NB)

        for _ in range(NB):
            wait_scatter()

    return k(xr, table)


def kernel(x, embeddings):
    b, h = x.shape
    _, d = embeddings.shape
    total = b * h
    b_per_w = total // NW
    nch = b_per_w // CH
    xr = x.reshape(NW, nch, CH).astype(jnp.int32)
    v = embeddings.shape[0]
    table_lin, hh = _transpose_table_tc(embeddings.T, v, d)
    out = _emb_lookup(xr, table_lin.reshape(2 * hh, d), total, d, nch)
    return out.reshape(b, h, d)
